# baseline (speedup 1.0000x reference)
import functools

import jax
import jax.numpy as jnp
from jax import lax
from jax.experimental import pallas as pl
from jax.experimental.pallas import tpu as pltpu
from jax.experimental.pallas import tpu_sc as plsc

N = 50000
E = 800000
HID = 64
CH = 128
EB = E // CH
NS = 16
CHK = 400
NCHK = N // CHK
CHK_ITERS = NCHK // NS + 1
K = 3
NM = 130
NT = NM // 2
TAIL = EB - NS * NM * K
ZCH = 80
ZNCHK = N // ZCH
ZIT = ZNCHK // NS + 1
R = 5000
G = N // R

_mesh = plsc.VectorSubcoreMesh(core_axis_name="c", subcore_axis_name="s")
_sc_params = pltpu.CompilerParams(use_tc_tiling_on_sc=False)


@functools.partial(
    pl.kernel,
    out_type=(jax.ShapeDtypeStruct((N, 16), jnp.float32),
              jax.ShapeDtypeStruct((N, 16), jnp.float32)),
    mesh=_mesh,
    compiler_params=_sc_params,
    scratch_types=[
        pltpu.VMEM((K, CH), jnp.int32),
        pltpu.VMEM((K, CH), jnp.int32),
        pltpu.VMEM((CH, 16), jnp.float32),
        pltpu.VMEM((CHK, 16), jnp.float32),
        pltpu.VMEM_SHARED((N, 16), jnp.float32),
        pltpu.SemaphoreType.DMA,
        pltpu.SemaphoreType.DMA,
        pltpu.SemaphoreType.DMA,
        pltpu.SemaphoreType.DMA,
    ],
)
def _hist(edges_hbm, deg_out_hbm, deg_in_hbm,
          idx_a, idx_b, ones_v, zero_v, slab,
          sem_sa, sem_sb, sem_ia, sem_ib):
    cc = lax.axis_index("c")
    ss = lax.axis_index("s")
    base = ss * NM * K

    @pl.loop(0, CH)
    def _(r):
        ones_v[r, pl.ds(0, 16)] = jnp.ones((16,), jnp.float32)

    @pl.loop(0, CHK)
    def _(r):
        zero_v[r, pl.ds(0, 16)] = jnp.zeros((16,), jnp.float32)

    @pl.loop(0, CHK_ITERS)
    def _(k):
        kk = ss + k * NS

        @pl.when(kk < NCHK)
        def _():
            pltpu.sync_copy(zero_v, slab.at[pl.ds(kk * CHK, CHK)])

    plsc.subcore_barrier()

    def _hist_half(which):
        def fire_sc(idx_ref, sem):
            for j in range(K):
                pltpu.async_copy(ones_v, slab.at[idx_ref.at[j]], sem, add=True)

        def drain_sc(idx_ref, sem):
            for j in range(K):
                pltpu.make_async_copy(ones_v, slab.at[idx_ref.at[j]], sem).wait()

        def fire_idx(m, idx_ref, sem):
            pltpu.async_copy(edges_hbm.at[which, pl.ds(base + m * K, K)],
                             idx_ref, sem)

        def wait_idx(idx_ref, sem):
            pltpu.make_async_copy(
                edges_hbm.at[which, pl.ds(0, K)], idx_ref, sem).wait()

        pltpu.sync_copy(edges_hbm.at[which, pl.ds(base, K)], idx_a)
        pltpu.sync_copy(edges_hbm.at[which, pl.ds(base + K, K)], idx_b)

        @pl.loop(0, NT)
        def _(t):
            fire_sc(idx_a, sem_sa)
            fire_sc(idx_b, sem_sb)
            drain_sc(idx_a, sem_sa)

            @pl.when(t < NT - 1)
            def _():
                fire_idx(2 * t + 2, idx_a, sem_ia)

            drain_sc(idx_b, sem_sb)

            @pl.when(t < NT - 1)
            def _():
                fire_idx(2 * t + 3, idx_b, sem_ib)
                wait_idx(idx_a, sem_ia)
                wait_idx(idx_b, sem_ib)

        @pl.when(ss < TAIL)
        def _():
            pltpu.sync_copy(edges_hbm.at[which, NS * NM * K + ss], idx_a.at[0])
            pltpu.sync_copy(ones_v, slab.at[idx_a.at[0]], add=True)

    @pl.when(cc == 0)
    def _():
        _hist_half(0)

    @pl.when(cc == 1)
    def _():
        _hist_half(1)

    plsc.subcore_barrier()

    @pl.loop(0, CHK_ITERS)
    def _(k):
        kk = ss + k * NS

        @pl.when(kk < NCHK)
        def _():
            @pl.when(cc == 0)
            def _():
                pltpu.sync_copy(slab.at[pl.ds(kk * CHK, CHK)],
                                deg_out_hbm.at[pl.ds(kk * CHK, CHK)])

            @pl.when(cc == 1)
            def _():
                pltpu.sync_copy(slab.at[pl.ds(kk * CHK, CHK)],
                                deg_in_hbm.at[pl.ds(kk * CHK, CHK)])


@functools.partial(
    pl.kernel,
    out_type=jax.ShapeDtypeStruct((N, HID), jnp.float32),
    mesh=_mesh,
    compiler_params=_sc_params,
    scratch_types=[
        pltpu.VMEM((K, CH), jnp.int32),
        pltpu.VMEM((K, CH), jnp.int32),
        pltpu.VMEM((K, CH), jnp.int32),
        pltpu.VMEM((K, CH), jnp.int32),
        pltpu.VMEM((K, CH, 32), jnp.float32),
        pltpu.VMEM((K, CH, 32), jnp.float32),
        pltpu.VMEM((ZCH, 32), jnp.float32),
        pltpu.VMEM_SHARED((N, 32), jnp.float32),
        pltpu.SemaphoreType.DMA,
        pltpu.SemaphoreType.DMA,
        pltpu.SemaphoreType.DMA,
        pltpu.SemaphoreType.DMA,
        pltpu.SemaphoreType.DMA,
        pltpu.SemaphoreType.DMA,
    ],
)
def _seg_sum(table_hbm, edges_hbm, out_agg,
             ids_a, idd_a, ids_b, idd_b, rows_a, rows_b, zero_v, slab,
             sem_ga, sem_gb, sem_sa, sem_sb, sem_ia, sem_ib):
    cc = lax.axis_index("c")
    ss = lax.axis_index("s")
    base = ss * NM * K

    @pl.loop(0, ZCH)
    def _(r):
        zero_v[r, pl.ds(0, 16)] = jnp.zeros((16,), jnp.float32)
        zero_v[r, pl.ds(16, 16)] = jnp.zeros((16,), jnp.float32)

    @pl.loop(0, ZIT)
    def _(k):
        kk = ss + k * NS

        @pl.when(kk < ZNCHK)
        def _():
            pltpu.sync_copy(zero_v, slab.at[pl.ds(kk * ZCH, ZCH)])

    plsc.subcore_barrier()

    def transform(ids_ref):
        for j in range(K):
            @pl.loop(0, CH, step=16)
            def _(v):
                ids_ref[j, pl.ds(v, 16)] = ids_ref[j, pl.ds(v, 16)] * 2 + cc

    def fire_idx(m, ids_ref, idd_ref, sem):
        pltpu.async_copy(edges_hbm.at[0, pl.ds(base + m * K, K)], ids_ref, sem)
        pltpu.async_copy(edges_hbm.at[1, pl.ds(base + m * K, K)], idd_ref, sem)

    def wait_idx(ids_ref, idd_ref, sem):
        pltpu.make_async_copy(
            edges_hbm.at[0, pl.ds(0, K)], ids_ref, sem).wait()
        pltpu.make_async_copy(
            edges_hbm.at[1, pl.ds(0, K)], idd_ref, sem).wait()

    def fire_g(ids_ref, rows_ref, sem):
        for j in range(K):
            pltpu.async_copy(table_hbm.at[ids_ref.at[j]], rows_ref.at[j], sem)

    def wait_g(ids_ref, rows_ref, sem):
        for j in range(K):
            pltpu.make_async_copy(
                table_hbm.at[ids_ref.at[j]], rows_ref.at[j], sem).wait()

    def fire_sc(rows_ref, idd_ref, sem):
        for j in range(K):
            pltpu.async_copy(rows_ref.at[j], slab.at[idd_ref.at[j]], sem,
                             add=True)

    def drain_sc(rows_ref, idd_ref, sem):
        for j in range(K):
            pltpu.make_async_copy(
                rows_ref.at[j], slab.at[idd_ref.at[j]], sem).wait()

    pltpu.sync_copy(edges_hbm.at[0, pl.ds(base, K)], ids_a)
    pltpu.sync_copy(edges_hbm.at[1, pl.ds(base, K)], idd_a)
    pltpu.sync_copy(edges_hbm.at[0, pl.ds(base + K, K)], ids_b)
    pltpu.sync_copy(edges_hbm.at[1, pl.ds(base + K, K)], idd_b)
    transform(ids_a)
    transform(ids_b)
    fire_g(ids_a, rows_a, sem_ga)
    fire_g(ids_b, rows_b, sem_gb)

    @pl.loop(0, NT)
    def _(t):
        wait_g(ids_a, rows_a, sem_ga)
        fire_sc(rows_a, idd_a, sem_sa)
        wait_g(ids_b, rows_b, sem_gb)
        fire_sc(rows_b, idd_b, sem_sb)
        drain_sc(rows_a, idd_a, sem_sa)

        @pl.when(t < NT - 1)
        def _():
            fire_idx(2 * t + 2, ids_a, idd_a, sem_ia)

        drain_sc(rows_b, idd_b, sem_sb)

        @pl.when(t < NT - 1)
        def _():
            fire_idx(2 * t + 3, ids_b, idd_b, sem_ib)
            wait_idx(ids_a, idd_a, sem_ia)
            transform(ids_a)
            fire_g(ids_a, rows_a, sem_ga)
            wait_idx(ids_b, idd_b, sem_ib)
            transform(ids_b)
            fire_g(ids_b, rows_b, sem_gb)

    @pl.when(ss < TAIL)
    def _():
        row = NS * NM * K + ss
        pltpu.sync_copy(edges_hbm.at[0, row], ids_a.at[0])
        pltpu.sync_copy(edges_hbm.at[1, row], idd_a.at[0])

        @pl.loop(0, CH, step=16)
        def _(v):
            ids_a[0, pl.ds(v, 16)] = ids_a[0, pl.ds(v, 16)] * 2 + cc

        pltpu.sync_copy(table_hbm.at[ids_a.at[0]], rows_a.at[0])
        pltpu.sync_copy(rows_a.at[0], slab.at[idd_a.at[0]], add=True)

    plsc.subcore_barrier()

    @pl.loop(0, CHK_ITERS)
    def _(k):
        kk = ss + k * NS

        @pl.when(kk < NCHK)
        def _():
            pltpu.sync_copy(
                slab.at[pl.ds(kk * CHK, CHK)],
                out_agg.at[pl.ds(kk * CHK, CHK), pl.ds(32 * cc, 32)])


def _mlp_body(h2_ref, h3_ref, W1_ref, W2a_ref, W2b_ref, b1_ref, b2_ref,
              pre_ref, s1_ref, s2_ref):
    i = pl.program_id(0)
    W2a = W2a_ref[...]
    Wa = jnp.dot(W1_ref[...], W2a, preferred_element_type=jnp.float32)
    cvec = jnp.dot(b1_ref[...], W2a,
                   preferred_element_type=jnp.float32) + b2_ref[...]
    pre = (jnp.dot(h2_ref[...], Wa, preferred_element_type=jnp.float32)
           + jnp.dot(h3_ref[...], W2b_ref[...],
                     preferred_element_type=jnp.float32)
           + cvec)
    pre_ref[...] = pre

    @pl.when(i == 0)
    def _():
        s1_ref[...] = jnp.zeros_like(s1_ref)
        s2_ref[...] = jnp.zeros_like(s2_ref)

    s1_ref[...] += jnp.sum(pre, axis=0, keepdims=True)
    s2_ref[...] += jnp.sum(pre * pre, axis=0, keepdims=True)


def _mlp(h2, h3, W1, W2a, W2b, b1r, b2r):
    return pl.pallas_call(
        _mlp_body,
        grid=(G,),
        in_specs=[
            pl.BlockSpec((R, 128), lambda i: (i, 0)),
            pl.BlockSpec((R, 128), lambda i: (i, 0)),
            pl.BlockSpec((128, 128), lambda i: (0, 0)),
            pl.BlockSpec((128, HID), lambda i: (0, 0)),
            pl.BlockSpec((128, HID), lambda i: (0, 0)),
            pl.BlockSpec((1, 128), lambda i: (0, 0)),
            pl.BlockSpec((1, HID), lambda i: (0, 0)),
        ],
        out_specs=[
            pl.BlockSpec((R, HID), lambda i: (i, 0)),
            pl.BlockSpec((1, HID), lambda i: (0, 0)),
            pl.BlockSpec((1, HID), lambda i: (0, 0)),
        ],
        out_shape=[
            jax.ShapeDtypeStruct((N, HID), jnp.float32),
            jax.ShapeDtypeStruct((1, HID), jnp.float32),
            jax.ShapeDtypeStruct((1, HID), jnp.float32),
        ],
    )(h2, h3, W1, W2a, W2b, b1r, b2r)


def _bn_scale_body(pre_ref, s1_ref, s2_ref, g_ref, b_ref, deg_ref, out_ref):
    m = s1_ref[...] / N
    v = s2_ref[...] / N - m * m
    inv = lax.rsqrt(v + 1e-5)
    hb = (pre_ref[...] - m) * inv * g_ref[...] + b_ref[...]
    scale = lax.rsqrt(jnp.maximum(deg_ref[:, 0:1], 1.0))
    out_ref[...] = hb * scale


def _bn_scale(pre, s1, s2, gr, br, deg16):
    return pl.pallas_call(
        _bn_scale_body,
        grid=(G,),
        in_specs=[
            pl.BlockSpec((R, HID), lambda i: (i, 0)),
            pl.BlockSpec((1, HID), lambda i: (0, 0)),
            pl.BlockSpec((1, HID), lambda i: (0, 0)),
            pl.BlockSpec((1, HID), lambda i: (0, 0)),
            pl.BlockSpec((1, HID), lambda i: (0, 0)),
            pl.BlockSpec((R, 16), lambda i: (i, 0)),
        ],
        out_specs=pl.BlockSpec((R, HID), lambda i: (i, 0)),
        out_shape=jax.ShapeDtypeStruct((N, HID), jnp.float32),
    )(pre, s1, s2, gr, br, deg16)


def _conv_bn_body(agg_ref, degi_ref, prev_ref, W_ref, b_ref,
                  g_ref, bt_ref, dego_ref, pre2_ref, hs_ref,
                  keep_ref, s1_ref, s2_ref):
    i = pl.program_id(0)

    @pl.when(i < G)
    def _():
        agg = agg_ref[...] * lax.rsqrt(jnp.maximum(degi_ref[:, 0:1], 1.0))
        y = prev_ref[...] + jnp.maximum(
            jnp.dot(agg, W_ref[...], preferred_element_type=jnp.float32)
            + b_ref[...], 0.0)
        pre2_ref[...] = y
        keep_ref[i] = y

        @pl.when(i == 0)
        def _():
            s1_ref[...] = jnp.zeros_like(s1_ref)
            s2_ref[...] = jnp.zeros_like(s2_ref)

        s1_ref[...] += jnp.sum(y, axis=0, keepdims=True)
        s2_ref[...] += jnp.sum(y * y, axis=0, keepdims=True)

    @pl.when(i >= G)
    def _():
        pre2 = keep_ref[i - G]
        m = s1_ref[...] / N
        v = s2_ref[...] / N - m * m
        inv = lax.rsqrt(v + 1e-5)
        hb = (pre2 - m) * inv * g_ref[...] + bt_ref[...]
        scale = lax.rsqrt(jnp.maximum(dego_ref[:, 0:1], 1.0))
        hs_ref[...] = hb * scale


def _conv_bn(agg, degi16, prev, W, br, gr, btr, dego16):
    lo = lambda i: (jnp.minimum(i, G - 1), 0)
    hi = lambda i: (jnp.maximum(i - G, 0), 0)
    zz = lambda i: (0, 0)
    return pl.pallas_call(
        _conv_bn_body,
        grid=(2 * G,),
        in_specs=[
            pl.BlockSpec((R, HID), lo),
            pl.BlockSpec((R, 16), lo),
            pl.BlockSpec((R, HID), lo),
            pl.BlockSpec((HID, HID), zz),
            pl.BlockSpec((1, HID), zz),
            pl.BlockSpec((1, HID), zz),
            pl.BlockSpec((1, HID), zz),
            pl.BlockSpec((R, 16), hi),
        ],
        out_specs=[
            pl.BlockSpec((R, HID), lo),
            pl.BlockSpec((R, HID), hi),
        ],
        out_shape=[
            jax.ShapeDtypeStruct((N, HID), jnp.float32),
            jax.ShapeDtypeStruct((N, HID), jnp.float32),
        ],
        scratch_shapes=[
            pltpu.VMEM((G, R, HID), jnp.float32),
            pltpu.VMEM((1, HID), jnp.float32),
            pltpu.VMEM((1, HID), jnp.float32),
        ],
    )(agg, degi16, prev, W, br, gr, btr, dego16)


def _final_body(agg_ref, deg_ref, pre_ref, W_ref, b_ref, ro_ref):
    i = pl.program_id(0)
    agg = agg_ref[...] * lax.rsqrt(jnp.maximum(deg_ref[:, 0:1], 1.0))
    y = pre_ref[...] + jnp.maximum(
        jnp.dot(agg, W_ref[...], preferred_element_type=jnp.float32)
        + b_ref[...], 0.0)

    @pl.when(i == 0)
    def _():
        ro_ref[...] = jnp.zeros_like(ro_ref)

    ro_ref[...] += jnp.sum(y, axis=0, keepdims=True)

    @pl.when(i == G - 1)
    def _():
        ro_ref[...] = ro_ref[...] / N


def _final(agg, degw, pre, W, br):
    return pl.pallas_call(
        _final_body,
        grid=(G,),
        in_specs=[
            pl.BlockSpec((R, HID), lambda i: (i, 0)),
            pl.BlockSpec((R, 16), lambda i: (i, 0)),
            pl.BlockSpec((R, HID), lambda i: (i, 0)),
            pl.BlockSpec((HID, HID), lambda i: (0, 0)),
            pl.BlockSpec((1, HID), lambda i: (0, 0)),
        ],
        out_specs=pl.BlockSpec((1, HID), lambda i: (0, 0)),
        out_shape=jax.ShapeDtypeStruct((1, HID), jnp.float32),
    )(agg, degw, pre, W, br)


def kernel(h1, h2, h3, edge_index, W1, b1, W2, b2,
           gamma1, beta1, gamma2, beta2, cW1, cb1, cW2, cb2):
    del h1
    e3d = edge_index.reshape(2, EB, CH)
    W2a = W2[:128]
    W2b = W2[128:]
    b1r = b1.reshape(1, 128)
    b2r = b2.reshape(1, HID)
    g1r = gamma1.reshape(1, HID)
    bt1r = beta1.reshape(1, HID)
    g2r = gamma2.reshape(1, HID)
    bt2r = beta2.reshape(1, HID)
    cb1r = cb1.reshape(1, HID)
    cb2r = cb2.reshape(1, HID)

    deg_out16, deg_in16 = _hist(e3d)
    pre, s1, s2 = _mlp(h2, h3, W1, W2a, W2b, b1r, b2r)
    hs1 = _bn_scale(pre, s1, s2, g1r, bt1r, deg_out16)
    a1 = _seg_sum(hs1.reshape(2 * N, 32), e3d)
    pre2, hs2 = _conv_bn(a1, deg_in16, pre, cW1, cb1r, g2r, bt2r, deg_out16)
    a2 = _seg_sum(hs2.reshape(2 * N, 32), e3d)
    return _final(a2, deg_in16, pre2, cW2, cb2r)


# speedup vs baseline: 1.0068x; 1.0068x over previous
import functools

import jax
import jax.numpy as jnp
from jax import lax
from jax.experimental import pallas as pl
from jax.experimental.pallas import tpu as pltpu
from jax.experimental.pallas import tpu_sc as plsc

N = 50000
E = 800000
HID = 64
CH = 128
EB = E // CH
NS = 16
CHK = 400
NCHK = N // CHK
CHK_ITERS = NCHK // NS + 1
K = 3
NM = 130
NT = NM // 2
TAIL = EB - NS * NM * K
ZCH = 80
ZNCHK = N // ZCH
ZIT = ZNCHK // NS + 1
R = 5000
G = N // R

_mesh = plsc.VectorSubcoreMesh(core_axis_name="c", subcore_axis_name="s")
_sc_params = pltpu.CompilerParams(use_tc_tiling_on_sc=False)


@functools.partial(
    pl.kernel,
    out_type=(jax.ShapeDtypeStruct((N, 16), jnp.float32),
              jax.ShapeDtypeStruct((N, 16), jnp.float32)),
    mesh=_mesh,
    compiler_params=_sc_params,
    scratch_types=[
        pltpu.VMEM((K, CH), jnp.int32),
        pltpu.VMEM((K, CH), jnp.int32),
        pltpu.VMEM((CH, 16), jnp.float32),
        pltpu.VMEM((CHK, 16), jnp.float32),
        pltpu.VMEM_SHARED((N, 16), jnp.float32),
        pltpu.SemaphoreType.DMA,
        pltpu.SemaphoreType.DMA,
        pltpu.SemaphoreType.DMA,
        pltpu.SemaphoreType.DMA,
    ],
)
def _hist(edges_hbm, deg_out_hbm, deg_in_hbm,
          idx_a, idx_b, ones_v, zero_v, slab,
          sem_sa, sem_sb, sem_ia, sem_ib):
    cc = lax.axis_index("c")
    ss = lax.axis_index("s")
    base = ss * NM * K

    @pl.loop(0, CH)
    def _(r):
        ones_v[r, pl.ds(0, 16)] = jnp.ones((16,), jnp.float32)

    @pl.loop(0, CHK)
    def _(r):
        zero_v[r, pl.ds(0, 16)] = jnp.zeros((16,), jnp.float32)

    @pl.loop(0, CHK_ITERS)
    def _(k):
        kk = ss + k * NS

        @pl.when(kk < NCHK)
        def _():
            pltpu.async_copy(zero_v, slab.at[pl.ds(kk * CHK, CHK)], sem_ia)

    @pl.loop(0, CHK_ITERS)
    def _(k):
        kk = ss + k * NS

        @pl.when(kk < NCHK)
        def _():
            pltpu.make_async_copy(
                zero_v, slab.at[pl.ds(kk * CHK, CHK)], sem_ia).wait()

    plsc.subcore_barrier()

    def _hist_half(which):
        def fire_sc(idx_ref, sem):
            for j in range(K):
                pltpu.async_copy(ones_v, slab.at[idx_ref.at[j]], sem, add=True)

        def drain_sc(idx_ref, sem):
            for j in range(K):
                pltpu.make_async_copy(ones_v, slab.at[idx_ref.at[j]], sem).wait()

        def fire_idx(m, idx_ref, sem):
            pltpu.async_copy(edges_hbm.at[which, pl.ds(base + m * K, K)],
                             idx_ref, sem)

        def wait_idx(idx_ref, sem):
            pltpu.make_async_copy(
                edges_hbm.at[which, pl.ds(0, K)], idx_ref, sem).wait()

        pltpu.sync_copy(edges_hbm.at[which, pl.ds(base, K)], idx_a)
        pltpu.sync_copy(edges_hbm.at[which, pl.ds(base + K, K)], idx_b)

        @pl.loop(0, NT)
        def _(t):
            fire_sc(idx_a, sem_sa)
            fire_sc(idx_b, sem_sb)
            drain_sc(idx_a, sem_sa)

            @pl.when(t < NT - 1)
            def _():
                fire_idx(2 * t + 2, idx_a, sem_ia)

            drain_sc(idx_b, sem_sb)

            @pl.when(t < NT - 1)
            def _():
                fire_idx(2 * t + 3, idx_b, sem_ib)
                wait_idx(idx_a, sem_ia)
                wait_idx(idx_b, sem_ib)

        @pl.when(ss < TAIL)
        def _():
            pltpu.sync_copy(edges_hbm.at[which, NS * NM * K + ss], idx_a.at[0])
            pltpu.sync_copy(ones_v, slab.at[idx_a.at[0]], add=True)

    @pl.when(cc == 0)
    def _():
        _hist_half(0)

    @pl.when(cc == 1)
    def _():
        _hist_half(1)

    plsc.subcore_barrier()

    @pl.loop(0, CHK_ITERS)
    def _(k):
        kk = ss + k * NS

        @pl.when(kk < NCHK)
        def _():
            @pl.when(cc == 0)
            def _():
                pltpu.sync_copy(slab.at[pl.ds(kk * CHK, CHK)],
                                deg_out_hbm.at[pl.ds(kk * CHK, CHK)])

            @pl.when(cc == 1)
            def _():
                pltpu.sync_copy(slab.at[pl.ds(kk * CHK, CHK)],
                                deg_in_hbm.at[pl.ds(kk * CHK, CHK)])


@functools.partial(
    pl.kernel,
    out_type=jax.ShapeDtypeStruct((N, HID), jnp.float32),
    mesh=_mesh,
    compiler_params=_sc_params,
    scratch_types=[
        pltpu.VMEM((K, CH), jnp.int32),
        pltpu.VMEM((K, CH), jnp.int32),
        pltpu.VMEM((K, CH), jnp.int32),
        pltpu.VMEM((K, CH), jnp.int32),
        pltpu.VMEM((K, CH, 32), jnp.float32),
        pltpu.VMEM((K, CH, 32), jnp.float32),
        pltpu.VMEM((ZCH, 32), jnp.float32),
        pltpu.VMEM_SHARED((N, 32), jnp.float32),
        pltpu.SemaphoreType.DMA,
        pltpu.SemaphoreType.DMA,
        pltpu.SemaphoreType.DMA,
        pltpu.SemaphoreType.DMA,
        pltpu.SemaphoreType.DMA,
        pltpu.SemaphoreType.DMA,
    ],
)
def _seg_sum(table_hbm, edges_hbm, out_agg,
             ids_a, idd_a, ids_b, idd_b, rows_a, rows_b, zero_v, slab,
             sem_ga, sem_gb, sem_sa, sem_sb, sem_ia, sem_ib):
    cc = lax.axis_index("c")
    ss = lax.axis_index("s")
    base = ss * NM * K

    @pl.loop(0, ZCH)
    def _(r):
        zero_v[r, pl.ds(0, 16)] = jnp.zeros((16,), jnp.float32)
        zero_v[r, pl.ds(16, 16)] = jnp.zeros((16,), jnp.float32)

    @pl.loop(0, ZIT)
    def _(k):
        kk = ss + k * NS

        @pl.when(kk < ZNCHK)
        def _():
            pltpu.async_copy(zero_v, slab.at[pl.ds(kk * ZCH, ZCH)], sem_ia)

    @pl.loop(0, ZIT)
    def _(k):
        kk = ss + k * NS

        @pl.when(kk < ZNCHK)
        def _():
            pltpu.make_async_copy(
                zero_v, slab.at[pl.ds(kk * ZCH, ZCH)], sem_ia).wait()

    plsc.subcore_barrier()

    def transform(ids_ref):
        for j in range(K):
            @pl.loop(0, CH, step=16)
            def _(v):
                ids_ref[j, pl.ds(v, 16)] = ids_ref[j, pl.ds(v, 16)] * 2 + cc

    def fire_idx(m, ids_ref, idd_ref, sem):
        pltpu.async_copy(edges_hbm.at[0, pl.ds(base + m * K, K)], ids_ref, sem)
        pltpu.async_copy(edges_hbm.at[1, pl.ds(base + m * K, K)], idd_ref, sem)

    def wait_idx(ids_ref, idd_ref, sem):
        pltpu.make_async_copy(
            edges_hbm.at[0, pl.ds(0, K)], ids_ref, sem).wait()
        pltpu.make_async_copy(
            edges_hbm.at[1, pl.ds(0, K)], idd_ref, sem).wait()

    def fire_g(ids_ref, rows_ref, sem):
        for j in range(K):
            pltpu.async_copy(table_hbm.at[ids_ref.at[j]], rows_ref.at[j], sem)

    def wait_g(ids_ref, rows_ref, sem):
        for j in range(K):
            pltpu.make_async_copy(
                table_hbm.at[ids_ref.at[j]], rows_ref.at[j], sem).wait()

    def fire_sc(rows_ref, idd_ref, sem):
        for j in range(K):
            pltpu.async_copy(rows_ref.at[j], slab.at[idd_ref.at[j]], sem,
                             add=True)

    def drain_sc(rows_ref, idd_ref, sem):
        for j in range(K):
            pltpu.make_async_copy(
                rows_ref.at[j], slab.at[idd_ref.at[j]], sem).wait()

    pltpu.sync_copy(edges_hbm.at[0, pl.ds(base, K)], ids_a)
    pltpu.sync_copy(edges_hbm.at[1, pl.ds(base, K)], idd_a)
    pltpu.sync_copy(edges_hbm.at[0, pl.ds(base + K, K)], ids_b)
    pltpu.sync_copy(edges_hbm.at[1, pl.ds(base + K, K)], idd_b)
    transform(ids_a)
    transform(ids_b)
    fire_g(ids_a, rows_a, sem_ga)
    fire_g(ids_b, rows_b, sem_gb)

    @pl.loop(0, NT)
    def _(t):
        wait_g(ids_a, rows_a, sem_ga)
        fire_sc(rows_a, idd_a, sem_sa)
        wait_g(ids_b, rows_b, sem_gb)
        fire_sc(rows_b, idd_b, sem_sb)
        drain_sc(rows_a, idd_a, sem_sa)

        @pl.when(t < NT - 1)
        def _():
            fire_idx(2 * t + 2, ids_a, idd_a, sem_ia)

        drain_sc(rows_b, idd_b, sem_sb)

        @pl.when(t < NT - 1)
        def _():
            fire_idx(2 * t + 3, ids_b, idd_b, sem_ib)
            wait_idx(ids_a, idd_a, sem_ia)
            transform(ids_a)
            fire_g(ids_a, rows_a, sem_ga)
            wait_idx(ids_b, idd_b, sem_ib)
            transform(ids_b)
            fire_g(ids_b, rows_b, sem_gb)

    @pl.when(ss < TAIL)
    def _():
        row = NS * NM * K + ss
        pltpu.sync_copy(edges_hbm.at[0, row], ids_a.at[0])
        pltpu.sync_copy(edges_hbm.at[1, row], idd_a.at[0])

        @pl.loop(0, CH, step=16)
        def _(v):
            ids_a[0, pl.ds(v, 16)] = ids_a[0, pl.ds(v, 16)] * 2 + cc

        pltpu.sync_copy(table_hbm.at[ids_a.at[0]], rows_a.at[0])
        pltpu.sync_copy(rows_a.at[0], slab.at[idd_a.at[0]], add=True)

    plsc.subcore_barrier()

    @pl.loop(0, CHK_ITERS)
    def _(k):
        kk = ss + k * NS

        @pl.when(kk < NCHK)
        def _():
            pltpu.sync_copy(
                slab.at[pl.ds(kk * CHK, CHK)],
                out_agg.at[pl.ds(kk * CHK, CHK), pl.ds(32 * cc, 32)])


def _mlp_body(h2_ref, h3_ref, W1_ref, W2a_ref, W2b_ref, b1_ref, b2_ref,
              pre_ref, s1_ref, s2_ref):
    i = pl.program_id(0)
    W2a = W2a_ref[...]
    Wa = jnp.dot(W1_ref[...], W2a, preferred_element_type=jnp.float32)
    cvec = jnp.dot(b1_ref[...], W2a,
                   preferred_element_type=jnp.float32) + b2_ref[...]
    pre = (jnp.dot(h2_ref[...], Wa, preferred_element_type=jnp.float32)
           + jnp.dot(h3_ref[...], W2b_ref[...],
                     preferred_element_type=jnp.float32)
           + cvec)
    pre_ref[...] = pre

    @pl.when(i == 0)
    def _():
        s1_ref[...] = jnp.zeros_like(s1_ref)
        s2_ref[...] = jnp.zeros_like(s2_ref)

    s1_ref[...] += jnp.sum(pre, axis=0, keepdims=True)
    s2_ref[...] += jnp.sum(pre * pre, axis=0, keepdims=True)


def _mlp(h2, h3, W1, W2a, W2b, b1r, b2r):
    return pl.pallas_call(
        _mlp_body,
        grid=(G,),
        in_specs=[
            pl.BlockSpec((R, 128), lambda i: (i, 0)),
            pl.BlockSpec((R, 128), lambda i: (i, 0)),
            pl.BlockSpec((128, 128), lambda i: (0, 0)),
            pl.BlockSpec((128, HID), lambda i: (0, 0)),
            pl.BlockSpec((128, HID), lambda i: (0, 0)),
            pl.BlockSpec((1, 128), lambda i: (0, 0)),
            pl.BlockSpec((1, HID), lambda i: (0, 0)),
        ],
        out_specs=[
            pl.BlockSpec((R, HID), lambda i: (i, 0)),
            pl.BlockSpec((1, HID), lambda i: (0, 0)),
            pl.BlockSpec((1, HID), lambda i: (0, 0)),
        ],
        out_shape=[
            jax.ShapeDtypeStruct((N, HID), jnp.float32),
            jax.ShapeDtypeStruct((1, HID), jnp.float32),
            jax.ShapeDtypeStruct((1, HID), jnp.float32),
        ],
    )(h2, h3, W1, W2a, W2b, b1r, b2r)


def _bn_scale_body(pre_ref, s1_ref, s2_ref, g_ref, b_ref, deg_ref, out_ref):
    m = s1_ref[...] / N
    v = s2_ref[...] / N - m * m
    inv = lax.rsqrt(v + 1e-5)
    hb = (pre_ref[...] - m) * inv * g_ref[...] + b_ref[...]
    scale = lax.rsqrt(jnp.maximum(deg_ref[:, 0:1], 1.0))
    out_ref[...] = hb * scale


def _bn_scale(pre, s1, s2, gr, br, deg16):
    return pl.pallas_call(
        _bn_scale_body,
        grid=(G,),
        in_specs=[
            pl.BlockSpec((R, HID), lambda i: (i, 0)),
            pl.BlockSpec((1, HID), lambda i: (0, 0)),
            pl.BlockSpec((1, HID), lambda i: (0, 0)),
            pl.BlockSpec((1, HID), lambda i: (0, 0)),
            pl.BlockSpec((1, HID), lambda i: (0, 0)),
            pl.BlockSpec((R, 16), lambda i: (i, 0)),
        ],
        out_specs=pl.BlockSpec((R, HID), lambda i: (i, 0)),
        out_shape=jax.ShapeDtypeStruct((N, HID), jnp.float32),
    )(pre, s1, s2, gr, br, deg16)


def _conv_bn_body(agg_ref, degi_ref, prev_ref, W_ref, b_ref,
                  g_ref, bt_ref, dego_ref, pre2_ref, hs_ref,
                  keep_ref, s1_ref, s2_ref):
    i = pl.program_id(0)

    @pl.when(i < G)
    def _():
        agg = agg_ref[...] * lax.rsqrt(jnp.maximum(degi_ref[:, 0:1], 1.0))
        y = prev_ref[...] + jnp.maximum(
            jnp.dot(agg, W_ref[...], preferred_element_type=jnp.float32)
            + b_ref[...], 0.0)
        pre2_ref[...] = y
        keep_ref[i] = y

        @pl.when(i == 0)
        def _():
            s1_ref[...] = jnp.zeros_like(s1_ref)
            s2_ref[...] = jnp.zeros_like(s2_ref)

        s1_ref[...] += jnp.sum(y, axis=0, keepdims=True)
        s2_ref[...] += jnp.sum(y * y, axis=0, keepdims=True)

    @pl.when(i >= G)
    def _():
        pre2 = keep_ref[i - G]
        m = s1_ref[...] / N
        v = s2_ref[...] / N - m * m
        inv = lax.rsqrt(v + 1e-5)
        hb = (pre2 - m) * inv * g_ref[...] + bt_ref[...]
        scale = lax.rsqrt(jnp.maximum(dego_ref[:, 0:1], 1.0))
        hs_ref[...] = hb * scale


def _conv_bn(agg, degi16, prev, W, br, gr, btr, dego16):
    lo = lambda i: (jnp.minimum(i, G - 1), 0)
    hi = lambda i: (jnp.maximum(i - G, 0), 0)
    zz = lambda i: (0, 0)
    return pl.pallas_call(
        _conv_bn_body,
        grid=(2 * G,),
        in_specs=[
            pl.BlockSpec((R, HID), lo),
            pl.BlockSpec((R, 16), lo),
            pl.BlockSpec((R, HID), lo),
            pl.BlockSpec((HID, HID), zz),
            pl.BlockSpec((1, HID), zz),
            pl.BlockSpec((1, HID), zz),
            pl.BlockSpec((1, HID), zz),
            pl.BlockSpec((R, 16), hi),
        ],
        out_specs=[
            pl.BlockSpec((R, HID), lo),
            pl.BlockSpec((R, HID), hi),
        ],
        out_shape=[
            jax.ShapeDtypeStruct((N, HID), jnp.float32),
            jax.ShapeDtypeStruct((N, HID), jnp.float32),
        ],
        scratch_shapes=[
            pltpu.VMEM((G, R, HID), jnp.float32),
            pltpu.VMEM((1, HID), jnp.float32),
            pltpu.VMEM((1, HID), jnp.float32),
        ],
    )(agg, degi16, prev, W, br, gr, btr, dego16)


def _final_body(agg_ref, deg_ref, pre_ref, W_ref, b_ref, ro_ref):
    i = pl.program_id(0)
    agg = agg_ref[...] * lax.rsqrt(jnp.maximum(deg_ref[:, 0:1], 1.0))
    y = pre_ref[...] + jnp.maximum(
        jnp.dot(agg, W_ref[...], preferred_element_type=jnp.float32)
        + b_ref[...], 0.0)

    @pl.when(i == 0)
    def _():
        ro_ref[...] = jnp.zeros_like(ro_ref)

    ro_ref[...] += jnp.sum(y, axis=0, keepdims=True)

    @pl.when(i == G - 1)
    def _():
        ro_ref[...] = ro_ref[...] / N


def _final(agg, degw, pre, W, br):
    return pl.pallas_call(
        _final_body,
        grid=(G,),
        in_specs=[
            pl.BlockSpec((R, HID), lambda i: (i, 0)),
            pl.BlockSpec((R, 16), lambda i: (i, 0)),
            pl.BlockSpec((R, HID), lambda i: (i, 0)),
            pl.BlockSpec((HID, HID), lambda i: (0, 0)),
            pl.BlockSpec((1, HID), lambda i: (0, 0)),
        ],
        out_specs=pl.BlockSpec((1, HID), lambda i: (0, 0)),
        out_shape=jax.ShapeDtypeStruct((1, HID), jnp.float32),
    )(agg, degw, pre, W, br)


def kernel(h1, h2, h3, edge_index, W1, b1, W2, b2,
           gamma1, beta1, gamma2, beta2, cW1, cb1, cW2, cb2):
    del h1
    e3d = edge_index.reshape(2, EB, CH)
    W2a = W2[:128]
    W2b = W2[128:]
    b1r = b1.reshape(1, 128)
    b2r = b2.reshape(1, HID)
    g1r = gamma1.reshape(1, HID)
    bt1r = beta1.reshape(1, HID)
    g2r = gamma2.reshape(1, HID)
    bt2r = beta2.reshape(1, HID)
    cb1r = cb1.reshape(1, HID)
    cb2r = cb2.reshape(1, HID)

    deg_out16, deg_in16 = _hist(e3d)
    pre, s1, s2 = _mlp(h2, h3, W1, W2a, W2b, b1r, b2r)
    hs1 = _bn_scale(pre, s1, s2, g1r, bt1r, deg_out16)
    a1 = _seg_sum(hs1.reshape(2 * N, 32), e3d)
    pre2, hs2 = _conv_bn(a1, deg_in16, pre, cW1, cb1r, g2r, bt2r, deg_out16)
    a2 = _seg_sum(hs2.reshape(2 * N, 32), e3d)
    return _final(a2, deg_in16, pre2, cW2, cb2r)


# speedup vs baseline: 1.0074x; 1.0007x over previous
import functools

import jax
import jax.numpy as jnp
from jax import lax
from jax.experimental import pallas as pl
from jax.experimental.pallas import tpu as pltpu
from jax.experimental.pallas import tpu_sc as plsc

N = 50000
E = 800000
HID = 64
CH = 128
EB = E // CH
NS = 16
CHK = 400
NCHK = N // CHK
CHK_ITERS = NCHK // NS + 1
K = 3
NM = 130
NT = NM // 2
TAIL = EB - NS * NM * K
ZCH = 80
ZNCHK = N // ZCH
ZIT = ZNCHK // NS + 1
R = 5000
G = N // R

_mesh = plsc.VectorSubcoreMesh(core_axis_name="c", subcore_axis_name="s")
_sc_params = pltpu.CompilerParams(use_tc_tiling_on_sc=False)


@functools.partial(
    pl.kernel,
    out_type=(jax.ShapeDtypeStruct((N, 16), jnp.float32),
              jax.ShapeDtypeStruct((N, 16), jnp.float32)),
    mesh=_mesh,
    compiler_params=_sc_params,
    scratch_types=[
        pltpu.VMEM((K, CH), jnp.int32),
        pltpu.VMEM((K, CH), jnp.int32),
        pltpu.VMEM((CH, 16), jnp.float32),
        pltpu.VMEM((CHK, 16), jnp.float32),
        pltpu.VMEM_SHARED((N, 16), jnp.float32),
        pltpu.SemaphoreType.DMA,
        pltpu.SemaphoreType.DMA,
        pltpu.SemaphoreType.DMA,
        pltpu.SemaphoreType.DMA,
    ],
)
def _hist(edges_hbm, deg_out_hbm, deg_in_hbm,
          idx_a, idx_b, ones_v, zero_v, slab,
          sem_sa, sem_sb, sem_ia, sem_ib):
    cc = lax.axis_index("c")
    ss = lax.axis_index("s")
    base = ss * NM * K

    @pl.loop(0, CH)
    def _(r):
        ones_v[r, pl.ds(0, 16)] = jnp.ones((16,), jnp.float32)

    @pl.loop(0, CHK)
    def _(r):
        zero_v[r, pl.ds(0, 16)] = jnp.zeros((16,), jnp.float32)

    @pl.loop(0, CHK_ITERS)
    def _(k):
        kk = ss + k * NS

        @pl.when(kk < NCHK)
        def _():
            pltpu.async_copy(zero_v, slab.at[pl.ds(kk * CHK, CHK)], sem_ia)

    @pl.loop(0, CHK_ITERS)
    def _(k):
        kk = ss + k * NS

        @pl.when(kk < NCHK)
        def _():
            pltpu.make_async_copy(
                zero_v, slab.at[pl.ds(kk * CHK, CHK)], sem_ia).wait()

    plsc.subcore_barrier()

    def _hist_half(which):
        def fire_sc(idx_ref, sem):
            for j in range(K):
                pltpu.async_copy(ones_v, slab.at[idx_ref.at[j]], sem, add=True)

        def drain_sc(idx_ref, sem):
            for j in range(K):
                pltpu.make_async_copy(ones_v, slab.at[idx_ref.at[j]], sem).wait()

        def fire_idx(m, idx_ref, sem):
            pltpu.async_copy(edges_hbm.at[which, pl.ds(base + m * K, K)],
                             idx_ref, sem)

        def wait_idx(idx_ref, sem):
            pltpu.make_async_copy(
                edges_hbm.at[which, pl.ds(0, K)], idx_ref, sem).wait()

        pltpu.sync_copy(edges_hbm.at[which, pl.ds(base, K)], idx_a)
        pltpu.sync_copy(edges_hbm.at[which, pl.ds(base + K, K)], idx_b)

        @pl.loop(0, NT)
        def _(t):
            fire_sc(idx_a, sem_sa)
            fire_sc(idx_b, sem_sb)
            drain_sc(idx_a, sem_sa)

            @pl.when(t < NT - 1)
            def _():
                fire_idx(2 * t + 2, idx_a, sem_ia)

            drain_sc(idx_b, sem_sb)

            @pl.when(t < NT - 1)
            def _():
                fire_idx(2 * t + 3, idx_b, sem_ib)
                wait_idx(idx_a, sem_ia)
                wait_idx(idx_b, sem_ib)

        @pl.when(ss < TAIL)
        def _():
            pltpu.sync_copy(edges_hbm.at[which, NS * NM * K + ss], idx_a.at[0])
            pltpu.sync_copy(ones_v, slab.at[idx_a.at[0]], add=True)

    @pl.when(cc == 0)
    def _():
        _hist_half(0)

    @pl.when(cc == 1)
    def _():
        _hist_half(1)

    plsc.subcore_barrier()

    @pl.loop(0, CHK_ITERS)
    def _(k):
        kk = ss + k * NS

        @pl.when(kk < NCHK)
        def _():
            @pl.when(cc == 0)
            def _():
                pltpu.async_copy(slab.at[pl.ds(kk * CHK, CHK)],
                                 deg_out_hbm.at[pl.ds(kk * CHK, CHK)], sem_ib)

            @pl.when(cc == 1)
            def _():
                pltpu.async_copy(slab.at[pl.ds(kk * CHK, CHK)],
                                 deg_in_hbm.at[pl.ds(kk * CHK, CHK)], sem_ib)

    @pl.loop(0, CHK_ITERS)
    def _(k):
        kk = ss + k * NS

        @pl.when(kk < NCHK)
        def _():
            @pl.when(cc == 0)
            def _():
                pltpu.make_async_copy(
                    slab.at[pl.ds(kk * CHK, CHK)],
                    deg_out_hbm.at[pl.ds(kk * CHK, CHK)], sem_ib).wait()

            @pl.when(cc == 1)
            def _():
                pltpu.make_async_copy(
                    slab.at[pl.ds(kk * CHK, CHK)],
                    deg_in_hbm.at[pl.ds(kk * CHK, CHK)], sem_ib).wait()


@functools.partial(
    pl.kernel,
    out_type=jax.ShapeDtypeStruct((N, HID), jnp.float32),
    mesh=_mesh,
    compiler_params=_sc_params,
    scratch_types=[
        pltpu.VMEM((K, CH), jnp.int32),
        pltpu.VMEM((K, CH), jnp.int32),
        pltpu.VMEM((K, CH), jnp.int32),
        pltpu.VMEM((K, CH), jnp.int32),
        pltpu.VMEM((K, CH, 32), jnp.float32),
        pltpu.VMEM((K, CH, 32), jnp.float32),
        pltpu.VMEM((ZCH, 32), jnp.float32),
        pltpu.VMEM_SHARED((N, 32), jnp.float32),
        pltpu.SemaphoreType.DMA,
        pltpu.SemaphoreType.DMA,
        pltpu.SemaphoreType.DMA,
        pltpu.SemaphoreType.DMA,
        pltpu.SemaphoreType.DMA,
        pltpu.SemaphoreType.DMA,
    ],
)
def _seg_sum(table_hbm, edges_hbm, out_agg,
             ids_a, idd_a, ids_b, idd_b, rows_a, rows_b, zero_v, slab,
             sem_ga, sem_gb, sem_sa, sem_sb, sem_ia, sem_ib):
    cc = lax.axis_index("c")
    ss = lax.axis_index("s")
    base = ss * NM * K

    @pl.loop(0, ZCH)
    def _(r):
        zero_v[r, pl.ds(0, 16)] = jnp.zeros((16,), jnp.float32)
        zero_v[r, pl.ds(16, 16)] = jnp.zeros((16,), jnp.float32)

    @pl.loop(0, ZIT)
    def _(k):
        kk = ss + k * NS

        @pl.when(kk < ZNCHK)
        def _():
            pltpu.async_copy(zero_v, slab.at[pl.ds(kk * ZCH, ZCH)], sem_ia)

    @pl.loop(0, ZIT)
    def _(k):
        kk = ss + k * NS

        @pl.when(kk < ZNCHK)
        def _():
            pltpu.make_async_copy(
                zero_v, slab.at[pl.ds(kk * ZCH, ZCH)], sem_ia).wait()

    plsc.subcore_barrier()

    def transform(ids_ref):
        for j in range(K):
            @pl.loop(0, CH, step=16)
            def _(v):
                ids_ref[j, pl.ds(v, 16)] = ids_ref[j, pl.ds(v, 16)] * 2 + cc

    def fire_idx(m, ids_ref, idd_ref, sem):
        pltpu.async_copy(edges_hbm.at[0, pl.ds(base + m * K, K)], ids_ref, sem)
        pltpu.async_copy(edges_hbm.at[1, pl.ds(base + m * K, K)], idd_ref, sem)

    def wait_idx(ids_ref, idd_ref, sem):
        pltpu.make_async_copy(
            edges_hbm.at[0, pl.ds(0, K)], ids_ref, sem).wait()
        pltpu.make_async_copy(
            edges_hbm.at[1, pl.ds(0, K)], idd_ref, sem).wait()

    def fire_g(ids_ref, rows_ref, sem):
        for j in range(K):
            pltpu.async_copy(table_hbm.at[ids_ref.at[j]], rows_ref.at[j], sem)

    def wait_g(ids_ref, rows_ref, sem):
        for j in range(K):
            pltpu.make_async_copy(
                table_hbm.at[ids_ref.at[j]], rows_ref.at[j], sem).wait()

    def fire_sc(rows_ref, idd_ref, sem):
        for j in range(K):
            pltpu.async_copy(rows_ref.at[j], slab.at[idd_ref.at[j]], sem,
                             add=True)

    def drain_sc(rows_ref, idd_ref, sem):
        for j in range(K):
            pltpu.make_async_copy(
                rows_ref.at[j], slab.at[idd_ref.at[j]], sem).wait()

    pltpu.sync_copy(edges_hbm.at[0, pl.ds(base, K)], ids_a)
    pltpu.sync_copy(edges_hbm.at[1, pl.ds(base, K)], idd_a)
    pltpu.sync_copy(edges_hbm.at[0, pl.ds(base + K, K)], ids_b)
    pltpu.sync_copy(edges_hbm.at[1, pl.ds(base + K, K)], idd_b)
    transform(ids_a)
    transform(ids_b)
    fire_g(ids_a, rows_a, sem_ga)
    fire_g(ids_b, rows_b, sem_gb)

    @pl.loop(0, NT)
    def _(t):
        wait_g(ids_a, rows_a, sem_ga)
        fire_sc(rows_a, idd_a, sem_sa)
        wait_g(ids_b, rows_b, sem_gb)
        fire_sc(rows_b, idd_b, sem_sb)
        drain_sc(rows_a, idd_a, sem_sa)

        @pl.when(t < NT - 1)
        def _():
            fire_idx(2 * t + 2, ids_a, idd_a, sem_ia)

        drain_sc(rows_b, idd_b, sem_sb)

        @pl.when(t < NT - 1)
        def _():
            fire_idx(2 * t + 3, ids_b, idd_b, sem_ib)
            wait_idx(ids_a, idd_a, sem_ia)
            transform(ids_a)
            fire_g(ids_a, rows_a, sem_ga)
            wait_idx(ids_b, idd_b, sem_ib)
            transform(ids_b)
            fire_g(ids_b, rows_b, sem_gb)

    @pl.when(ss < TAIL)
    def _():
        row = NS * NM * K + ss
        pltpu.sync_copy(edges_hbm.at[0, row], ids_a.at[0])
        pltpu.sync_copy(edges_hbm.at[1, row], idd_a.at[0])

        @pl.loop(0, CH, step=16)
        def _(v):
            ids_a[0, pl.ds(v, 16)] = ids_a[0, pl.ds(v, 16)] * 2 + cc

        pltpu.sync_copy(table_hbm.at[ids_a.at[0]], rows_a.at[0])
        pltpu.sync_copy(rows_a.at[0], slab.at[idd_a.at[0]], add=True)

    plsc.subcore_barrier()

    @pl.loop(0, CHK_ITERS)
    def _(k):
        kk = ss + k * NS

        @pl.when(kk < NCHK)
        def _():
            pltpu.async_copy(
                slab.at[pl.ds(kk * CHK, CHK)],
                out_agg.at[pl.ds(kk * CHK, CHK), pl.ds(32 * cc, 32)], sem_ia)

    @pl.loop(0, CHK_ITERS)
    def _(k):
        kk = ss + k * NS

        @pl.when(kk < NCHK)
        def _():
            pltpu.make_async_copy(
                slab.at[pl.ds(kk * CHK, CHK)],
                out_agg.at[pl.ds(kk * CHK, CHK), pl.ds(32 * cc, 32)],
                sem_ia).wait()


def _mlp_body(h2_ref, h3_ref, W1_ref, W2a_ref, W2b_ref, b1_ref, b2_ref,
              pre_ref, s1_ref, s2_ref):
    i = pl.program_id(0)
    W2a = W2a_ref[...]
    Wa = jnp.dot(W1_ref[...], W2a, preferred_element_type=jnp.float32)
    cvec = jnp.dot(b1_ref[...], W2a,
                   preferred_element_type=jnp.float32) + b2_ref[...]
    pre = (jnp.dot(h2_ref[...], Wa, preferred_element_type=jnp.float32)
           + jnp.dot(h3_ref[...], W2b_ref[...],
                     preferred_element_type=jnp.float32)
           + cvec)
    pre_ref[...] = pre

    @pl.when(i == 0)
    def _():
        s1_ref[...] = jnp.zeros_like(s1_ref)
        s2_ref[...] = jnp.zeros_like(s2_ref)

    s1_ref[...] += jnp.sum(pre, axis=0, keepdims=True)
    s2_ref[...] += jnp.sum(pre * pre, axis=0, keepdims=True)


def _mlp(h2, h3, W1, W2a, W2b, b1r, b2r):
    return pl.pallas_call(
        _mlp_body,
        grid=(G,),
        in_specs=[
            pl.BlockSpec((R, 128), lambda i: (i, 0)),
            pl.BlockSpec((R, 128), lambda i: (i, 0)),
            pl.BlockSpec((128, 128), lambda i: (0, 0)),
            pl.BlockSpec((128, HID), lambda i: (0, 0)),
            pl.BlockSpec((128, HID), lambda i: (0, 0)),
            pl.BlockSpec((1, 128), lambda i: (0, 0)),
            pl.BlockSpec((1, HID), lambda i: (0, 0)),
        ],
        out_specs=[
            pl.BlockSpec((R, HID), lambda i: (i, 0)),
            pl.BlockSpec((1, HID), lambda i: (0, 0)),
            pl.BlockSpec((1, HID), lambda i: (0, 0)),
        ],
        out_shape=[
            jax.ShapeDtypeStruct((N, HID), jnp.float32),
            jax.ShapeDtypeStruct((1, HID), jnp.float32),
            jax.ShapeDtypeStruct((1, HID), jnp.float32),
        ],
    )(h2, h3, W1, W2a, W2b, b1r, b2r)


def _bn_scale_body(pre_ref, s1_ref, s2_ref, g_ref, b_ref, deg_ref, out_ref):
    m = s1_ref[...] / N
    v = s2_ref[...] / N - m * m
    inv = lax.rsqrt(v + 1e-5)
    hb = (pre_ref[...] - m) * inv * g_ref[...] + b_ref[...]
    scale = lax.rsqrt(jnp.maximum(deg_ref[:, 0:1], 1.0))
    out_ref[...] = hb * scale


def _bn_scale(pre, s1, s2, gr, br, deg16):
    return pl.pallas_call(
        _bn_scale_body,
        grid=(G,),
        in_specs=[
            pl.BlockSpec((R, HID), lambda i: (i, 0)),
            pl.BlockSpec((1, HID), lambda i: (0, 0)),
            pl.BlockSpec((1, HID), lambda i: (0, 0)),
            pl.BlockSpec((1, HID), lambda i: (0, 0)),
            pl.BlockSpec((1, HID), lambda i: (0, 0)),
            pl.BlockSpec((R, 16), lambda i: (i, 0)),
        ],
        out_specs=pl.BlockSpec((R, HID), lambda i: (i, 0)),
        out_shape=jax.ShapeDtypeStruct((N, HID), jnp.float32),
    )(pre, s1, s2, gr, br, deg16)


def _conv_bn_body(agg_ref, degi_ref, prev_ref, W_ref, b_ref,
                  g_ref, bt_ref, dego_ref, pre2_ref, hs_ref,
                  keep_ref, s1_ref, s2_ref):
    i = pl.program_id(0)

    @pl.when(i < G)
    def _():
        agg = agg_ref[...] * lax.rsqrt(jnp.maximum(degi_ref[:, 0:1], 1.0))
        y = prev_ref[...] + jnp.maximum(
            jnp.dot(agg, W_ref[...], preferred_element_type=jnp.float32)
            + b_ref[...], 0.0)
        pre2_ref[...] = y
        keep_ref[i] = y

        @pl.when(i == 0)
        def _():
            s1_ref[...] = jnp.zeros_like(s1_ref)
            s2_ref[...] = jnp.zeros_like(s2_ref)

        s1_ref[...] += jnp.sum(y, axis=0, keepdims=True)
        s2_ref[...] += jnp.sum(y * y, axis=0, keepdims=True)

    @pl.when(i >= G)
    def _():
        pre2 = keep_ref[i - G]
        m = s1_ref[...] / N
        v = s2_ref[...] / N - m * m
        inv = lax.rsqrt(v + 1e-5)
        hb = (pre2 - m) * inv * g_ref[...] + bt_ref[...]
        scale = lax.rsqrt(jnp.maximum(dego_ref[:, 0:1], 1.0))
        hs_ref[...] = hb * scale


def _conv_bn(agg, degi16, prev, W, br, gr, btr, dego16):
    lo = lambda i: (jnp.minimum(i, G - 1), 0)
    hi = lambda i: (jnp.maximum(i - G, 0), 0)
    zz = lambda i: (0, 0)
    return pl.pallas_call(
        _conv_bn_body,
        grid=(2 * G,),
        in_specs=[
            pl.BlockSpec((R, HID), lo),
            pl.BlockSpec((R, 16), lo),
            pl.BlockSpec((R, HID), lo),
            pl.BlockSpec((HID, HID), zz),
            pl.BlockSpec((1, HID), zz),
            pl.BlockSpec((1, HID), zz),
            pl.BlockSpec((1, HID), zz),
            pl.BlockSpec((R, 16), hi),
        ],
        out_specs=[
            pl.BlockSpec((R, HID), lo),
            pl.BlockSpec((R, HID), hi),
        ],
        out_shape=[
            jax.ShapeDtypeStruct((N, HID), jnp.float32),
            jax.ShapeDtypeStruct((N, HID), jnp.float32),
        ],
        scratch_shapes=[
            pltpu.VMEM((G, R, HID), jnp.float32),
            pltpu.VMEM((1, HID), jnp.float32),
            pltpu.VMEM((1, HID), jnp.float32),
        ],
    )(agg, degi16, prev, W, br, gr, btr, dego16)


def _final_body(agg_ref, deg_ref, pre_ref, W_ref, b_ref, ro_ref):
    i = pl.program_id(0)
    agg = agg_ref[...] * lax.rsqrt(jnp.maximum(deg_ref[:, 0:1], 1.0))
    y = pre_ref[...] + jnp.maximum(
        jnp.dot(agg, W_ref[...], preferred_element_type=jnp.float32)
        + b_ref[...], 0.0)

    @pl.when(i == 0)
    def _():
        ro_ref[...] = jnp.zeros_like(ro_ref)

    ro_ref[...] += jnp.sum(y, axis=0, keepdims=True)

    @pl.when(i == G - 1)
    def _():
        ro_ref[...] = ro_ref[...] / N


def _final(agg, degw, pre, W, br):
    return pl.pallas_call(
        _final_body,
        grid=(G,),
        in_specs=[
            pl.BlockSpec((R, HID), lambda i: (i, 0)),
            pl.BlockSpec((R, 16), lambda i: (i, 0)),
            pl.BlockSpec((R, HID), lambda i: (i, 0)),
            pl.BlockSpec((HID, HID), lambda i: (0, 0)),
            pl.BlockSpec((1, HID), lambda i: (0, 0)),
        ],
        out_specs=pl.BlockSpec((1, HID), lambda i: (0, 0)),
        out_shape=jax.ShapeDtypeStruct((1, HID), jnp.float32),
    )(agg, degw, pre, W, br)


def kernel(h1, h2, h3, edge_index, W1, b1, W2, b2,
           gamma1, beta1, gamma2, beta2, cW1, cb1, cW2, cb2):
    del h1
    e3d = edge_index.reshape(2, EB, CH)
    W2a = W2[:128]
    W2b = W2[128:]
    b1r = b1.reshape(1, 128)
    b2r = b2.reshape(1, HID)
    g1r = gamma1.reshape(1, HID)
    bt1r = beta1.reshape(1, HID)
    g2r = gamma2.reshape(1, HID)
    bt2r = beta2.reshape(1, HID)
    cb1r = cb1.reshape(1, HID)
    cb2r = cb2.reshape(1, HID)

    deg_out16, deg_in16 = _hist(e3d)
    pre, s1, s2 = _mlp(h2, h3, W1, W2a, W2b, b1r, b2r)
    hs1 = _bn_scale(pre, s1, s2, g1r, bt1r, deg_out16)
    a1 = _seg_sum(hs1.reshape(2 * N, 32), e3d)
    pre2, hs2 = _conv_bn(a1, deg_in16, pre, cW1, cb1r, g2r, bt2r, deg_out16)
    a2 = _seg_sum(hs2.reshape(2 * N, 32), e3d)
    return _final(a2, deg_in16, pre2, cW2, cb2r)
